# Initial kernel scaffold; baseline (speedup 1.0000x reference)
import jax
import jax.numpy as jnp
from jax.experimental import pallas as pl


def kernel(x, W1, W2, Ws1, Ws2, Wg):
    raise NotImplementedError("write your pallas kernel here")


# speedup vs baseline: 1.6211x; 1.6211x over previous
import jax
import jax.numpy as jnp
from jax.experimental import pallas as pl
from jax.experimental.pallas import tpu as pltpu

N, D, E, F = 2048, 1024, 16, 512
ET = E + 2


def _moe_body(x_ref, wg_ref, w1_ref, w2_ref, out_ref, wfull_ref):
    e = pl.program_id(0)

    @pl.when(e == 0)
    def _gating():
        x = x_ref[...]
        logits = jax.lax.dot_general(
            x, wg_ref[...], (((1,), (1,)), ((), ())),
            preferred_element_type=jnp.float32)
        iota = jax.lax.broadcasted_iota(jnp.int32, (N, E), 1)
        l0 = jnp.max(logits, axis=1, keepdims=True)
        m0 = logits == l0
        i0 = jnp.min(jnp.where(m0, iota, E), axis=1, keepdims=True)
        a0 = iota == i0
        masked = jnp.where(a0, -jnp.inf, logits)
        l1 = jnp.max(masked, axis=1, keepdims=True)
        m1 = masked == l1
        i1 = jnp.min(jnp.where(m1, iota, E), axis=1, keepdims=True)
        a1 = iota == i1
        w0 = jax.nn.sigmoid(l0 - l1)
        w1 = 1.0 - w0
        wdense = a0 * w0 + a1 * w1
        wfull_ref[:, :E] = wdense
        wfull_ref[:, E:] = jnp.ones((N, 32 - E), jnp.float32)

    x = x_ref[...]
    h = jax.lax.dot_general(
        x, w1_ref[0], (((1,), (1,)), ((), ())),
        preferred_element_type=jnp.float32)
    h = h * jax.nn.sigmoid(h)
    y = jax.lax.dot_general(
        h, w2_ref[0], (((1,), (1,)), ((), ())),
        preferred_element_type=jnp.float32)
    onehot = (jax.lax.broadcasted_iota(jnp.int32, (32, 1), 0) == e).astype(jnp.float32)
    w = jax.lax.dot_general(
        wfull_ref[...], onehot, (((1,), (0,)), ((), ())),
        preferred_element_type=jnp.float32)

    @pl.when(e == 0)
    def _init():
        out_ref[...] = jnp.zeros_like(out_ref)

    out_ref[...] += y * w


def kernel(x, W1, W2, Ws1, Ws2, Wg):
    W1e = jnp.concatenate([W1, Ws1.reshape(2, F, D)], axis=0)
    Ws2s = jnp.stack([Ws2[:, :F], Ws2[:, F:]], axis=0)
    W2e = jnp.concatenate([W2, Ws2s], axis=0)

    out = pl.pallas_call(
        _moe_body,
        grid=(ET,),
        in_specs=[
            pl.BlockSpec((N, D), lambda e: (0, 0)),
            pl.BlockSpec((E, D), lambda e: (0, 0)),
            pl.BlockSpec((1, F, D), lambda e: (e, 0, 0)),
            pl.BlockSpec((1, D, F), lambda e: (e, 0, 0)),
        ],
        out_specs=pl.BlockSpec((N, D), lambda e: (0, 0)),
        out_shape=jax.ShapeDtypeStruct((N, D), jnp.float32),
        scratch_shapes=[pltpu.VMEM((N, 32), jnp.float32)],
    )(x, Wg, W1e, W2e)
    return out
